# baseline (device time: 20443 ns/iter reference)
import jax
import jax.numpy as jnp
from jax import lax
from jax.experimental import pallas as pl
from jax.experimental.pallas import tpu as pltpu

N_DEV = 4


def kernel(x, router_W, route_idx, expert_W, shared_W):
    n_tok, d_model = x.shape
    n_local_exp, _, d_ff = expert_W.shape
    n_exp = router_W.shape[1]

    def body(x_ref, rw_ref, idx_ref, ew_ref, sw_ref, out_ref,
             comm_ref, send_sems, recv_sems):
        my_pos = lax.axis_index("i")
        left = lax.rem(my_pos + N_DEV - 1, N_DEV)
        right = lax.rem(my_pos + 1, N_DEV)

        comm_ref[0, :, :, :] = ew_ref[:, :, :]

        barrier_sem = pltpu.get_barrier_semaphore()
        for nbr in (left, right):
            pl.semaphore_signal(barrier_sem, inc=1, device_id=(nbr,),
                                device_id_type=pl.DeviceIdType.MESH)
        pl.semaphore_wait(barrier_sem, 2)

        def hop(h):
            return pltpu.make_async_remote_copy(
                src_ref=comm_ref.at[h],
                dst_ref=comm_ref.at[h + 1],
                send_sem=send_sems.at[h],
                recv_sem=recv_sems.at[h + 1],
                device_id=(right,),
                device_id_type=pl.DeviceIdType.MESH,
            )

        rdma0 = hop(0)
        rdma0.start()

        xv = x_ref[...]
        eid = idx_ref[...]
        scores = jnp.dot(xv, rw_ref[...], preferred_element_type=jnp.float32)
        m = jnp.max(scores, axis=-1, keepdims=True)
        p = jnp.exp(scores - m)
        denom = jnp.sum(p, axis=-1, keepdims=True)
        onehot = lax.broadcasted_iota(jnp.int32, (n_tok, n_exp), 1) == eid
        gate = jnp.sum(jnp.where(onehot, p, 0.0), axis=-1, keepdims=True) / denom

        acc = jnp.dot(xv, sw_ref[...], preferred_element_type=jnp.float32)

        def chunk_contrib(slot, acc):
            origin = lax.rem(my_pos + N_DEV - slot, N_DEV)
            for j in range(n_local_exp):
                e = origin * n_local_exp + j
                w = jnp.where(eid == e, gate, 0.0)
                acc = acc + w * jnp.dot(xv, comm_ref[slot, j],
                                        preferred_element_type=jnp.float32)
            return acc

        acc = chunk_contrib(0, acc)
        rdma0.wait()

        rdma1 = hop(1)
        rdma1.start()
        acc = chunk_contrib(1, acc)
        rdma1.wait()

        rdma2 = hop(2)
        rdma2.start()
        acc = chunk_contrib(2, acc)
        rdma2.wait()

        acc = chunk_contrib(3, acc)
        out_ref[...] = acc

    return pl.pallas_call(
        body,
        out_shape=jax.ShapeDtypeStruct((n_tok, d_ff), jnp.float32),
        in_specs=[pl.BlockSpec(memory_space=pltpu.VMEM)] * 5,
        out_specs=pl.BlockSpec(memory_space=pltpu.VMEM),
        scratch_shapes=[
            pltpu.VMEM((N_DEV, n_local_exp, d_model, d_ff), jnp.float32),
            pltpu.SemaphoreType.DMA((N_DEV,)),
            pltpu.SemaphoreType.DMA((N_DEV,)),
        ],
        compiler_params=pltpu.CompilerParams(collective_id=0),
    )(x, router_W, route_idx, expert_W, shared_W)


# device time: 14895 ns/iter; 1.3725x vs baseline; 1.3725x over previous
import jax
import jax.numpy as jnp
from jax import lax
from jax.experimental import pallas as pl
from jax.experimental.pallas import tpu as pltpu

N_DEV = 4


def kernel(x, router_W, route_idx, expert_W, shared_W):
    n_tok, d_model = x.shape
    n_local_exp, _, d_ff = expert_W.shape
    n_exp = router_W.shape[1]

    def body(x_ref, rw_ref, idx_ref, ew_ref, sw_ref, out_ref,
             comm_ref, send_sems, recv_sems):
        my_pos = lax.axis_index("i")

        barrier_sem = pltpu.get_barrier_semaphore()
        for k in range(1, N_DEV):
            pl.semaphore_signal(
                barrier_sem, inc=1,
                device_id=(lax.rem(my_pos + k, N_DEV),),
                device_id_type=pl.DeviceIdType.MESH,
            )
        pl.semaphore_wait(barrier_sem, N_DEV - 1)

        sends = []
        for k in range(1, N_DEV):
            s = pltpu.make_async_remote_copy(
                src_ref=ew_ref,
                dst_ref=comm_ref.at[N_DEV - k],
                send_sem=send_sems.at[k - 1],
                recv_sem=recv_sems.at[N_DEV - k],
                device_id=(lax.rem(my_pos + k, N_DEV),),
                device_id_type=pl.DeviceIdType.MESH,
            )
            s.start()
            sends.append(s)

        def recv_for(slot):
            return pltpu.make_async_remote_copy(
                src_ref=ew_ref,
                dst_ref=comm_ref.at[slot],
                send_sem=send_sems.at[N_DEV - 1],
                recv_sem=recv_sems.at[slot],
                device_id=(my_pos,),
                device_id_type=pl.DeviceIdType.MESH,
            )

        xv = x_ref[...]
        eid = idx_ref[...]
        scores = jnp.dot(xv, rw_ref[...], preferred_element_type=jnp.float32)
        m = jnp.max(scores, axis=-1, keepdims=True)
        p = jnp.exp(scores - m)
        denom = jnp.sum(p, axis=-1, keepdims=True)
        onehot = lax.broadcasted_iota(jnp.int32, (n_tok, n_exp), 1) == eid
        gate = jnp.sum(jnp.where(onehot, p, 0.0), axis=-1, keepdims=True) / denom

        acc = jnp.dot(xv, sw_ref[...], preferred_element_type=jnp.float32)

        def contrib(origin, w_ref2d_list, acc):
            for j, Wj in enumerate(w_ref2d_list):
                e = origin * n_local_exp + j
                w = jnp.where(eid == e, gate, 0.0)
                acc = acc + w * jnp.dot(xv, Wj,
                                        preferred_element_type=jnp.float32)
            return acc

        acc = contrib(my_pos, [ew_ref[j] for j in range(n_local_exp)], acc)

        for slot in (1, 3, 2):
            recv_for(slot).wait_recv()
            origin = lax.rem(my_pos + slot, N_DEV)
            acc = contrib(origin,
                          [comm_ref[slot, j] for j in range(n_local_exp)],
                          acc)

        for s in sends:
            s.wait_send()

        out_ref[...] = acc

    return pl.pallas_call(
        body,
        out_shape=jax.ShapeDtypeStruct((n_tok, d_ff), jnp.float32),
        in_specs=[pl.BlockSpec(memory_space=pltpu.VMEM)] * 5,
        out_specs=pl.BlockSpec(memory_space=pltpu.VMEM),
        scratch_shapes=[
            pltpu.VMEM((N_DEV, n_local_exp, d_model, d_ff), jnp.float32),
            pltpu.SemaphoreType.DMA((N_DEV,)),
            pltpu.SemaphoreType.DMA((N_DEV,)),
        ],
        compiler_params=pltpu.CompilerParams(collective_id=0),
    )(x, router_W, route_idx, expert_W, shared_W)


# device time: 12072 ns/iter; 1.6934x vs baseline; 1.2338x over previous
import jax
import jax.numpy as jnp
from jax import lax
from jax.experimental import pallas as pl
from jax.experimental.pallas import tpu as pltpu

N_DEV = 4


def kernel(x, router_W, route_idx, expert_W, shared_W):
    n_tok, d_model = x.shape
    n_local_exp, _, d_ff = expert_W.shape
    n_exp = router_W.shape[1]

    def body(x_ref, rw_ref, idx_ref, ew_ref, sw_ref, out_ref,
             ew16_ref, comm_ref, send_sems, recv_sems):
        my_pos = lax.axis_index("i")

        ew16_ref[...] = ew_ref[...].astype(jnp.bfloat16)

        barrier_sem = pltpu.get_barrier_semaphore()
        for k in range(1, N_DEV):
            pl.semaphore_signal(
                barrier_sem, inc=1,
                device_id=(lax.rem(my_pos + k, N_DEV),),
                device_id_type=pl.DeviceIdType.MESH,
            )
        pl.semaphore_wait(barrier_sem, N_DEV - 1)

        sends = []
        for k in range(1, N_DEV):
            s = pltpu.make_async_remote_copy(
                src_ref=ew16_ref,
                dst_ref=comm_ref.at[N_DEV - k],
                send_sem=send_sems.at[k - 1],
                recv_sem=recv_sems.at[N_DEV - k],
                device_id=(lax.rem(my_pos + k, N_DEV),),
                device_id_type=pl.DeviceIdType.MESH,
            )
            s.start()
            sends.append(s)

        def recv_for(slot):
            return pltpu.make_async_remote_copy(
                src_ref=ew16_ref,
                dst_ref=comm_ref.at[slot],
                send_sem=send_sems.at[N_DEV - 1],
                recv_sem=recv_sems.at[slot],
                device_id=(my_pos,),
                device_id_type=pl.DeviceIdType.MESH,
            )

        xv = x_ref[...]
        eid = idx_ref[...]
        scores = jnp.dot(xv, rw_ref[...], preferred_element_type=jnp.float32)
        m = jnp.max(scores, axis=-1, keepdims=True)
        p = jnp.exp(scores - m)
        denom = jnp.sum(p, axis=-1, keepdims=True)
        onehot = lax.broadcasted_iota(jnp.int32, (n_tok, n_exp), 1) == eid
        gate = jnp.sum(jnp.where(onehot, p, 0.0), axis=-1, keepdims=True) / denom

        x16 = xv.astype(jnp.bfloat16)
        acc = jnp.dot(x16, sw_ref[...].astype(jnp.bfloat16),
                      preferred_element_type=jnp.float32)

        def contrib(origin, w_ref2d_list, acc):
            for j, Wj in enumerate(w_ref2d_list):
                e = origin * n_local_exp + j
                w = jnp.where(eid == e, gate, 0.0)
                acc = acc + w * jnp.dot(x16, Wj,
                                        preferred_element_type=jnp.float32)
            return acc

        acc = contrib(my_pos, [ew16_ref[j] for j in range(n_local_exp)], acc)

        for slot in (1, 3, 2):
            recv_for(slot).wait_recv()
            origin = lax.rem(my_pos + slot, N_DEV)
            acc = contrib(origin,
                          [comm_ref[slot, j] for j in range(n_local_exp)],
                          acc)

        for s in sends:
            s.wait_send()

        out_ref[...] = acc

    return pl.pallas_call(
        body,
        out_shape=jax.ShapeDtypeStruct((n_tok, d_ff), jnp.float32),
        in_specs=[pl.BlockSpec(memory_space=pltpu.VMEM)] * 5,
        out_specs=pl.BlockSpec(memory_space=pltpu.VMEM),
        scratch_shapes=[
            pltpu.VMEM((n_local_exp, d_model, d_ff), jnp.bfloat16),
            pltpu.VMEM((N_DEV, n_local_exp, d_model, d_ff), jnp.bfloat16),
            pltpu.SemaphoreType.DMA((N_DEV,)),
            pltpu.SemaphoreType.DMA((N_DEV,)),
        ],
        compiler_params=pltpu.CompilerParams(collective_id=0),
    )(x, router_W, route_idx, expert_W, shared_W)


# device time: 11890 ns/iter; 1.7193x vs baseline; 1.0153x over previous
import jax
import jax.numpy as jnp
from jax import lax
from jax.experimental import pallas as pl
from jax.experimental.pallas import tpu as pltpu

N_DEV = 4


def kernel(x, router_W, route_idx, expert_W, shared_W):
    n_tok, d_model = x.shape
    n_local_exp, _, d_ff = expert_W.shape
    n_exp = router_W.shape[1]
    d_cat = n_local_exp * d_model

    def body(x_ref, rw_ref, idx_ref, ew_ref, sw_ref, out_ref,
             ew16_ref, comm_ref, send_sems, recv_sems):
        my_pos = lax.axis_index("i")

        barrier_sem = pltpu.get_barrier_semaphore()
        for k in range(1, N_DEV):
            pl.semaphore_signal(
                barrier_sem, inc=1,
                device_id=(lax.rem(my_pos + k, N_DEV),),
                device_id_type=pl.DeviceIdType.MESH,
            )
        ew16_ref[...] = (
            ew_ref[...].reshape(d_cat, d_ff).astype(jnp.bfloat16)
        )
        pl.semaphore_wait(barrier_sem, N_DEV - 1)

        sends = []
        for k in range(1, N_DEV):
            s = pltpu.make_async_remote_copy(
                src_ref=ew16_ref,
                dst_ref=comm_ref.at[N_DEV - k],
                send_sem=send_sems.at[k - 1],
                recv_sem=recv_sems.at[N_DEV - k],
                device_id=(lax.rem(my_pos + k, N_DEV),),
                device_id_type=pl.DeviceIdType.MESH,
            )
            s.start()
            sends.append(s)

        def recv_for(slot):
            return pltpu.make_async_remote_copy(
                src_ref=ew16_ref,
                dst_ref=comm_ref.at[slot],
                send_sem=send_sems.at[N_DEV - 1],
                recv_sem=recv_sems.at[slot],
                device_id=(my_pos,),
                device_id_type=pl.DeviceIdType.MESH,
            )

        xv = x_ref[...]
        eid = idx_ref[...]
        scores = jnp.dot(xv, rw_ref[...], preferred_element_type=jnp.float32)
        m = jnp.max(scores, axis=-1, keepdims=True)
        p = jnp.exp(scores - m)
        denom = jnp.sum(p, axis=-1, keepdims=True)
        onehot = lax.broadcasted_iota(jnp.int32, (n_tok, n_exp), 1) == eid
        gate = jnp.sum(jnp.where(onehot, p, 0.0), axis=-1, keepdims=True) / denom

        def scaled_x_for(origin):
            parts = []
            for j in range(n_local_exp):
                e = origin * n_local_exp + j
                w = jnp.where(eid == e, gate, 0.0)
                parts.append((xv * w).astype(jnp.bfloat16))
            return jnp.concatenate(parts, axis=1)

        xm = [scaled_x_for(lax.rem(my_pos + s, N_DEV)) for s in range(N_DEV)]

        x16 = xv.astype(jnp.bfloat16)
        acc = jnp.dot(x16, sw_ref[...].astype(jnp.bfloat16),
                      preferred_element_type=jnp.float32)
        acc = acc + jnp.dot(xm[0], ew16_ref[...],
                            preferred_element_type=jnp.float32)

        for slot in (1, 3, 2):
            recv_for(slot).wait_recv()
            acc = acc + jnp.dot(xm[slot], comm_ref[slot],
                                preferred_element_type=jnp.float32)

        for s in sends:
            s.wait_send()

        out_ref[...] = acc

    return pl.pallas_call(
        body,
        out_shape=jax.ShapeDtypeStruct((n_tok, d_ff), jnp.float32),
        in_specs=[pl.BlockSpec(memory_space=pltpu.VMEM)] * 5,
        out_specs=pl.BlockSpec(memory_space=pltpu.VMEM),
        scratch_shapes=[
            pltpu.VMEM((d_cat, d_ff), jnp.bfloat16),
            pltpu.VMEM((N_DEV, d_cat, d_ff), jnp.bfloat16),
            pltpu.SemaphoreType.DMA((N_DEV,)),
            pltpu.SemaphoreType.DMA((N_DEV,)),
        ],
        compiler_params=pltpu.CompilerParams(collective_id=0),
    )(x, router_W, route_idx, expert_W, shared_W)
